# baseline (speedup 1.0000x reference)
import functools

import jax
import jax.numpy as jnp
from jax import lax
from jax.experimental import pallas as pl
from jax.experimental.pallas import tpu as pltpu
from jax.experimental.pallas import tpu_sc as plsc

F = 32
L = 16
NC, NS = 2, 16
NW = NC * NS
B = 16384
ROWS_W = B // NW
CH = 128
NCH = ROWS_W // CH


def _sc_scores(uid, iid, eid, Wu_inv, Wi_inv, Wu_env, Wi_env, W_env, cls_W, cls_b16):
    mesh = plsc.VectorSubcoreMesh(core_axis_name="c", subcore_axis_name="s")
    out_type = [jax.ShapeDtypeStruct((B,), jnp.float32) for _ in range(4)]
    scratch_types = [
        pltpu.VMEM((NCH, CH), jnp.int32),
        pltpu.VMEM((NCH, CH), jnp.int32),
        pltpu.VMEM((NCH, CH), jnp.int32),
        pltpu.VMEM((ROWS_W, F), jnp.float32),
        pltpu.VMEM((ROWS_W, F), jnp.float32),
        pltpu.VMEM((ROWS_W, F), jnp.float32),
        pltpu.VMEM((ROWS_W, F), jnp.float32),
        pltpu.VMEM((ROWS_W,), jnp.float32),
        pltpu.VMEM((ROWS_W,), jnp.float32),
        pltpu.VMEM((ROWS_W,), jnp.float32),
        pltpu.VMEM((ROWS_W,), jnp.float32),
        pltpu.VMEM((2, F), jnp.float32),
        pltpu.VMEM((2, F), jnp.float32),
        pltpu.VMEM((L,), jnp.float32),
        pltpu.SemaphoreType.DMA,
    ]

    @functools.partial(pl.kernel, mesh=mesh, out_type=out_type,
                       scratch_types=scratch_types,
                       compiler_params=pltpu.CompilerParams(
                           needs_layout_passes=False,
                           use_tc_tiling_on_sc=False))
    def body(uid_h, iid_h, eid_h, wui_h, wii_h, wue_h, wie_h, wev_h, clsw_h,
             clsb_h, o_inv_h, o_env_h, o_l0_h, o_l1_h,
             idx_u, idx_i, idx_e, r_ui, r_ii, r_ue, r_ie,
             res_inv, res_env, res_l0, res_l1, clsw_v, wenv_v, clsb_v, sem):
        wid = lax.axis_index("s") * NC + lax.axis_index("c")
        base_row = wid * NCH
        pltpu.sync_copy(uid_h.at[pl.ds(base_row, NCH)], idx_u)
        pltpu.sync_copy(iid_h.at[pl.ds(base_row, NCH)], idx_i)
        pltpu.sync_copy(eid_h.at[pl.ds(base_row, NCH)], idx_e)
        pltpu.sync_copy(clsw_h, clsw_v)
        pltpu.sync_copy(wev_h, wenv_v)
        pltpu.sync_copy(clsb_h, clsb_v)

        copies = []
        for j in range(NCH):
            dst = pl.ds(j * CH, CH)
            copies.append([
                pltpu.async_copy(wui_h.at[idx_u.at[j]], r_ui.at[dst], sem),
                pltpu.async_copy(wii_h.at[idx_i.at[j]], r_ii.at[dst], sem),
                pltpu.async_copy(wue_h.at[idx_u.at[j]], r_ue.at[dst], sem),
                pltpu.async_copy(wie_h.at[idx_i.at[j]], r_ie.at[dst], sem),
            ])

        cw = [clsw_v[0, pl.ds(0, L)], clsw_v[0, pl.ds(L, L)],
              clsw_v[1, pl.ds(0, L)], clsw_v[1, pl.ds(L, L)]]
        we = [wenv_v[0, pl.ds(0, L)], wenv_v[0, pl.ds(L, L)],
              wenv_v[1, pl.ds(0, L)], wenv_v[1, pl.ds(L, L)]]
        w0 = [cw[f // L][f % L] for f in range(F)]
        w1 = [cw[2 + f // L][f % L] for f in range(F)]
        we0 = [we[f // L][f % L] for f in range(F)]
        we1 = [we[2 + f // L][f % L] for f in range(F)]
        vb = clsb_v[...]
        b0 = vb[0]
        b1 = vb[1]
        lanes = lax.iota(jnp.int32, L)
        zero = jnp.zeros((L,), jnp.float32)

        for j in range(NCH):
            for c in copies[j]:
                c.wait()

            def kblock(kb, _, j=j):
                base_r = j * CH + kb * L
                rows = base_r + lanes
                env_is0 = idx_e[j, pl.ds(kb * L, L)] == 0
                acc_inv = zero
                acc_env = zero
                acc_l0 = zero
                acc_l1 = zero
                for f in range(F):
                    cf = jnp.full((L,), f, jnp.int32)
                    gu = plsc.load_gather(r_ui, [rows, cf])
                    gi = plsc.load_gather(r_ii, [rows, cf])
                    p = gu * gi
                    acc_inv = acc_inv + p
                    acc_l0 = acc_l0 + p * w0[f]
                    acc_l1 = acc_l1 + p * w1[f]
                    eu = plsc.load_gather(r_ue, [rows, cf])
                    ei = plsc.load_gather(r_ie, [rows, cf])
                    ee = jnp.where(env_is0, we0[f], we1[f])
                    acc_env = acc_env + eu * ei * ee
                sl = pl.ds(base_r, L)
                res_inv[sl] = acc_inv
                res_env[sl] = acc_inv + acc_env
                res_l0[sl] = acc_l0 + b0
                res_l1[sl] = acc_l1 + b1
                return 0

            lax.fori_loop(0, CH // L, kblock, 0)

        base = wid * ROWS_W
        pltpu.sync_copy(res_inv, o_inv_h.at[pl.ds(base, ROWS_W)])
        pltpu.sync_copy(res_env, o_env_h.at[pl.ds(base, ROWS_W)])
        pltpu.sync_copy(res_l0, o_l0_h.at[pl.ds(base, ROWS_W)])
        pltpu.sync_copy(res_l1, o_l1_h.at[pl.ds(base, ROWS_W)])

    return body(uid, iid, eid, Wu_inv, Wi_inv, Wu_env, Wi_env, W_env,
                cls_W, cls_b16)


def _tc_log_softmax2(l0_ref, l1_ref, o0_ref, o1_ref):
    a = l0_ref[...]
    b = l1_ref[...]
    m = jnp.maximum(a, b)
    lse = m + jnp.log(jnp.exp(a - m) + jnp.exp(b - m))
    o0_ref[...] = a - lse
    o1_ref[...] = b - lse


def kernel(users_id, items_id, envs_id, alpha, Wu_inv, Wi_inv, Wu_env, Wi_env,
           W_env, cls_W, cls_b):
    del alpha
    uid = users_id.astype(jnp.int32).reshape(NW * NCH, CH)
    iid = items_id.astype(jnp.int32).reshape(NW * NCH, CH)
    eid = envs_id.astype(jnp.int32).reshape(NW * NCH, CH)
    cls_b16 = jnp.zeros((L,), jnp.float32).at[:2].set(cls_b.astype(jnp.float32))

    inv_score, env_score, l0, l1 = _sc_scores(
        uid, iid, eid, Wu_inv, Wi_inv, Wu_env, Wi_env, W_env,
        cls_W.astype(jnp.float32), cls_b16)

    o0, o1 = pl.pallas_call(
        _tc_log_softmax2,
        out_shape=[jax.ShapeDtypeStruct((CH, CH), jnp.float32)] * 2,
    )(l0.reshape(CH, CH), l1.reshape(CH, CH))

    env_outputs = jnp.stack([o0.reshape(-1), o1.reshape(-1)], axis=-1)
    return inv_score, env_score, env_outputs


# speedup vs baseline: 1.0005x; 1.0005x over previous
import functools

import jax
import jax.numpy as jnp
from jax import lax
from jax.experimental import pallas as pl
from jax.experimental.pallas import tpu as pltpu
from jax.experimental.pallas import tpu_sc as plsc

F = 32
L = 16
NC, NS = 2, 16
NW = NC * NS
B = 16384
ROWS_W = B // NW
CH = 64
NCH = ROWS_W // CH
RPB = 128 // F
NBLK = 1000000 // RPB


def _sc_scores(uid, iid, eid, Wu_inv, Wi_inv, Wu_env, Wi_env, W_env, cls_W,
               cls_b16):
    mesh = plsc.VectorSubcoreMesh(core_axis_name="c", subcore_axis_name="s")
    out_type = [jax.ShapeDtypeStruct((B,), jnp.float32) for _ in range(4)]
    scratch_types = [
        pltpu.VMEM((NCH, CH), jnp.int32),
        pltpu.VMEM((NCH, CH), jnp.int32),
        pltpu.VMEM((NCH, CH), jnp.int32),
        pltpu.VMEM((NCH, CH), jnp.int32),
        pltpu.VMEM((NCH, CH), jnp.int32),
        pltpu.VMEM((2, CH, 128), jnp.float32),
        pltpu.VMEM((2, CH, 128), jnp.float32),
        pltpu.VMEM((2, CH, 128), jnp.float32),
        pltpu.VMEM((2, CH, 128), jnp.float32),
        pltpu.VMEM((ROWS_W,), jnp.float32),
        pltpu.VMEM((ROWS_W,), jnp.float32),
        pltpu.VMEM((ROWS_W,), jnp.float32),
        pltpu.VMEM((ROWS_W,), jnp.float32),
        pltpu.VMEM((2, F), jnp.float32),
        pltpu.VMEM((2, F), jnp.float32),
        pltpu.VMEM((L,), jnp.float32),
        pltpu.SemaphoreType.DMA,
        pltpu.SemaphoreType.DMA,
    ]

    @functools.partial(pl.kernel, mesh=mesh, out_type=out_type,
                       scratch_types=scratch_types,
                       compiler_params=pltpu.CompilerParams(
                           needs_layout_passes=False,
                           use_tc_tiling_on_sc=False))
    def body(uid_h, iid_h, eid_h, wui_h, wii_h, wue_h, wie_h, wev_h, clsw_h,
             clsb_h, o_inv_h, o_env_h, o_l0_h, o_l1_h,
             ids_u, ids_i, ids_e, blk_u, blk_i,
             d_ui, d_ii, d_ue, d_ie,
             res_inv, res_env, res_l0, res_l1, clsw_v, wenv_v, clsb_v,
             sem0, sem1):
        wid = lax.axis_index("s") * NC + lax.axis_index("c")
        base_row = wid * NCH
        pltpu.sync_copy(uid_h.at[pl.ds(base_row, NCH)], ids_u)
        pltpu.sync_copy(iid_h.at[pl.ds(base_row, NCH)], ids_i)
        pltpu.sync_copy(eid_h.at[pl.ds(base_row, NCH)], ids_e)
        pltpu.sync_copy(clsw_h, clsw_v)
        pltpu.sync_copy(wev_h, wenv_v)
        pltpu.sync_copy(clsb_h, clsb_v)

        for j in range(NCH):
            def gen(kb, _, j=j):
                sl = pl.ds(kb * L, L)
                blk_u[j, sl] = ids_u[j, sl] >> 2
                blk_i[j, sl] = ids_i[j, sl] >> 2
                return 0
            lax.fori_loop(0, CH // L, gen, 0)

        sems = [sem0, sem1]
        copies = [None] * NCH

        def fire(j):
            s = j % 2
            copies[j] = [
                pltpu.async_copy(wui_h.at[blk_u.at[j]], d_ui.at[s], sems[s]),
                pltpu.async_copy(wii_h.at[blk_i.at[j]], d_ii.at[s], sems[s]),
                pltpu.async_copy(wue_h.at[blk_u.at[j]], d_ue.at[s], sems[s]),
                pltpu.async_copy(wie_h.at[blk_i.at[j]], d_ie.at[s], sems[s]),
            ]

        fire(0)
        fire(1)

        cw = [clsw_v[0, pl.ds(0, L)], clsw_v[0, pl.ds(L, L)],
              clsw_v[1, pl.ds(0, L)], clsw_v[1, pl.ds(L, L)]]
        we = [wenv_v[0, pl.ds(0, L)], wenv_v[0, pl.ds(L, L)],
              wenv_v[1, pl.ds(0, L)], wenv_v[1, pl.ds(L, L)]]
        w0 = [cw[f // L][f % L] for f in range(F)]
        w1 = [cw[2 + f // L][f % L] for f in range(F)]
        we0 = [we[f // L][f % L] for f in range(F)]
        we1 = [we[2 + f // L][f % L] for f in range(F)]
        vb = clsb_v[...]
        b0 = vb[0]
        b1 = vb[1]
        lanes = lax.iota(jnp.int32, L)
        zero = jnp.zeros((L,), jnp.float32)

        for j in range(NCH):
            for c in copies[j]:
                c.wait()
            s = j % 2
            vu = d_ui.at[s]
            vi = d_ii.at[s]
            vue = d_ue.at[s]
            vie = d_ie.at[s]

            def kblock(kb, _, j=j, s=s, vu=vu, vi=vi, vue=vue, vie=vie):
                eb = kb * L
                sl = pl.ds(eb, L)
                env_is0 = ids_e[j, sl] == 0
                ucol = (ids_u[j, sl] & 3) << 5
                icol = (ids_i[j, sl] & 3) << 5
                rows = eb + lanes
                acc_inv = zero
                acc_env = zero
                acc_l0 = zero
                acc_l1 = zero
                for f in range(F):
                    cu = ucol + f
                    ci = icol + f
                    gu = plsc.load_gather(vu, [rows, cu])
                    gi = plsc.load_gather(vi, [rows, ci])
                    p = gu * gi
                    acc_inv = acc_inv + p
                    acc_l0 = acc_l0 + p * w0[f]
                    acc_l1 = acc_l1 + p * w1[f]
                    eu = plsc.load_gather(vue, [rows, cu])
                    ei = plsc.load_gather(vie, [rows, ci])
                    ee = jnp.where(env_is0, we0[f], we1[f])
                    acc_env = acc_env + eu * ei * ee
                out = pl.ds(j * CH + eb, L)
                res_inv[out] = acc_inv
                res_env[out] = acc_inv + acc_env
                res_l0[out] = acc_l0 + b0
                res_l1[out] = acc_l1 + b1
                return 0

            lax.fori_loop(0, CH // L, kblock, 0)
            if j + 2 < NCH:
                fire(j + 2)

        base = wid * ROWS_W
        pltpu.sync_copy(res_inv, o_inv_h.at[pl.ds(base, ROWS_W)])
        pltpu.sync_copy(res_env, o_env_h.at[pl.ds(base, ROWS_W)])
        pltpu.sync_copy(res_l0, o_l0_h.at[pl.ds(base, ROWS_W)])
        pltpu.sync_copy(res_l1, o_l1_h.at[pl.ds(base, ROWS_W)])

    return body(uid, iid, eid, Wu_inv, Wi_inv, Wu_env, Wi_env, W_env,
                cls_W, cls_b16)


def _tc_log_softmax2(l0_ref, l1_ref, o0_ref, o1_ref):
    a = l0_ref[...]
    b = l1_ref[...]
    m = jnp.maximum(a, b)
    lse = m + jnp.log(jnp.exp(a - m) + jnp.exp(b - m))
    o0_ref[...] = a - lse
    o1_ref[...] = b - lse


def kernel(users_id, items_id, envs_id, alpha, Wu_inv, Wi_inv, Wu_env, Wi_env,
           W_env, cls_W, cls_b):
    del alpha
    uid = users_id.astype(jnp.int32).reshape(NW * NCH, CH)
    iid = items_id.astype(jnp.int32).reshape(NW * NCH, CH)
    eid = envs_id.astype(jnp.int32).reshape(NW * NCH, CH)
    cls_b16 = jnp.zeros((L,), jnp.float32).at[:2].set(cls_b.astype(jnp.float32))

    inv_score, env_score, l0, l1 = _sc_scores(
        uid, iid, eid,
        Wu_inv.reshape(NBLK, 128), Wi_inv.reshape(NBLK, 128),
        Wu_env.reshape(NBLK, 128), Wi_env.reshape(NBLK, 128),
        W_env, cls_W.astype(jnp.float32), cls_b16)

    o0, o1 = pl.pallas_call(
        _tc_log_softmax2,
        out_shape=[jax.ShapeDtypeStruct((128, 128), jnp.float32)] * 2,
    )(l0.reshape(128, 128), l1.reshape(128, 128))

    env_outputs = jnp.stack([o0.reshape(-1), o1.reshape(-1)], axis=-1)
    return inv_score, env_score, env_outputs


# speedup vs baseline: 1.0018x; 1.0013x over previous
import functools

import jax
import jax.numpy as jnp
from jax import lax
from jax.experimental import pallas as pl
from jax.experimental.pallas import tpu as pltpu
from jax.experimental.pallas import tpu_sc as plsc

F = 32
L = 16
NC, NS = 2, 16
NW = NC * NS
B = 16384
ROWS_W = B // NW
CH = 64
NCH = ROWS_W // CH
RPB = 128 // F
NBLK = 1000000 // RPB


def _sc_scores(uid, iid, eid, Wu_inv, Wi_inv, Wu_env, Wi_env, W_env, cls_W,
               cls_b16):
    mesh = plsc.VectorSubcoreMesh(core_axis_name="c", subcore_axis_name="s")
    out_type = [jax.ShapeDtypeStruct((B,), jnp.float32) for _ in range(4)]
    scratch_types = [
        pltpu.VMEM((NCH, CH), jnp.int32),
        pltpu.VMEM((NCH, CH), jnp.int32),
        pltpu.VMEM((NCH, CH), jnp.int32),
        pltpu.VMEM((NCH, CH), jnp.int32),
        pltpu.VMEM((NCH, CH), jnp.int32),
        pltpu.VMEM((2, CH, 128), jnp.float32),
        pltpu.VMEM((2, CH, 128), jnp.float32),
        pltpu.VMEM((2, CH, 128), jnp.float32),
        pltpu.VMEM((2, CH, 128), jnp.float32),
        pltpu.VMEM((ROWS_W,), jnp.float32),
        pltpu.VMEM((ROWS_W,), jnp.float32),
        pltpu.VMEM((ROWS_W,), jnp.float32),
        pltpu.VMEM((ROWS_W,), jnp.float32),
        pltpu.VMEM((2, F), jnp.float32),
        pltpu.VMEM((2, F), jnp.float32),
        pltpu.VMEM((L,), jnp.float32),
        pltpu.SemaphoreType.DMA,
        pltpu.SemaphoreType.DMA,
    ]

    @functools.partial(pl.kernel, mesh=mesh, out_type=out_type,
                       scratch_types=scratch_types,
                       compiler_params=pltpu.CompilerParams(
                           needs_layout_passes=False,
                           use_tc_tiling_on_sc=True))
    def body(uid_h, iid_h, eid_h, wui_h, wii_h, wue_h, wie_h, wev_h, clsw_h,
             clsb_h, o_inv_h, o_env_h, o_l0_h, o_l1_h,
             ids_u, ids_i, ids_e, blk_u, blk_i,
             d_ui, d_ii, d_ue, d_ie,
             res_inv, res_env, res_l0, res_l1, clsw_v, wenv_v, clsb_v,
             sem0, sem1):
        wid = lax.axis_index("s") * NC + lax.axis_index("c")
        base_row = wid * NCH
        pltpu.sync_copy(uid_h.at[pl.ds(base_row, NCH)], ids_u)
        pltpu.sync_copy(iid_h.at[pl.ds(base_row, NCH)], ids_i)
        pltpu.sync_copy(eid_h.at[pl.ds(base_row, NCH)], ids_e)
        pltpu.sync_copy(clsw_h, clsw_v)
        pltpu.sync_copy(wev_h, wenv_v)
        pltpu.sync_copy(clsb_h, clsb_v)

        for j in range(NCH):
            def gen(kb, _, j=j):
                sl = pl.ds(kb * L, L)
                blk_u[j, sl] = ids_u[j, sl] >> 2
                blk_i[j, sl] = ids_i[j, sl] >> 2
                return 0
            lax.fori_loop(0, CH // L, gen, 0)

        sems = [sem0, sem1]
        copies = [None] * NCH

        def fire(j):
            s = j % 2
            copies[j] = [
                pltpu.async_copy(wui_h.at[blk_u.at[j]], d_ui.at[s], sems[s]),
                pltpu.async_copy(wii_h.at[blk_i.at[j]], d_ii.at[s], sems[s]),
                pltpu.async_copy(wue_h.at[blk_u.at[j]], d_ue.at[s], sems[s]),
                pltpu.async_copy(wie_h.at[blk_i.at[j]], d_ie.at[s], sems[s]),
            ]

        fire(0)
        fire(1)

        cw = [clsw_v[0, pl.ds(0, L)], clsw_v[0, pl.ds(L, L)],
              clsw_v[1, pl.ds(0, L)], clsw_v[1, pl.ds(L, L)]]
        we = [wenv_v[0, pl.ds(0, L)], wenv_v[0, pl.ds(L, L)],
              wenv_v[1, pl.ds(0, L)], wenv_v[1, pl.ds(L, L)]]
        w0 = [cw[f // L][f % L] for f in range(F)]
        w1 = [cw[2 + f // L][f % L] for f in range(F)]
        we0 = [we[f // L][f % L] for f in range(F)]
        we1 = [we[2 + f // L][f % L] for f in range(F)]
        vb = clsb_v[...]
        b0 = vb[0]
        b1 = vb[1]
        lanes = lax.iota(jnp.int32, L)
        zero = jnp.zeros((L,), jnp.float32)

        for j in range(NCH):
            for c in copies[j]:
                c.wait()
            s = j % 2
            vu = d_ui.at[s]
            vi = d_ii.at[s]
            vue = d_ue.at[s]
            vie = d_ie.at[s]

            def kblock(kb, _, j=j, s=s, vu=vu, vi=vi, vue=vue, vie=vie):
                eb = kb * L
                sl = pl.ds(eb, L)
                env_is0 = ids_e[j, sl] == 0
                ucol = (ids_u[j, sl] & 3) << 5
                icol = (ids_i[j, sl] & 3) << 5
                rows = eb + lanes
                acc_inv = zero
                acc_env = zero
                acc_l0 = zero
                acc_l1 = zero
                for f in range(F):
                    cu = ucol + f
                    ci = icol + f
                    gu = plsc.load_gather(vu, [rows, cu])
                    gi = plsc.load_gather(vi, [rows, ci])
                    p = gu * gi
                    acc_inv = acc_inv + p
                    acc_l0 = acc_l0 + p * w0[f]
                    acc_l1 = acc_l1 + p * w1[f]
                    eu = plsc.load_gather(vue, [rows, cu])
                    ei = plsc.load_gather(vie, [rows, ci])
                    ee = jnp.where(env_is0, we0[f], we1[f])
                    acc_env = acc_env + eu * ei * ee
                out = pl.ds(j * CH + eb, L)
                res_inv[out] = acc_inv
                res_env[out] = acc_inv + acc_env
                res_l0[out] = acc_l0 + b0
                res_l1[out] = acc_l1 + b1
                return 0

            lax.fori_loop(0, CH // L, kblock, 0)
            if j + 2 < NCH:
                fire(j + 2)

        base = wid * ROWS_W
        pltpu.sync_copy(res_inv, o_inv_h.at[pl.ds(base, ROWS_W)])
        pltpu.sync_copy(res_env, o_env_h.at[pl.ds(base, ROWS_W)])
        pltpu.sync_copy(res_l0, o_l0_h.at[pl.ds(base, ROWS_W)])
        pltpu.sync_copy(res_l1, o_l1_h.at[pl.ds(base, ROWS_W)])

    return body(uid, iid, eid, Wu_inv, Wi_inv, Wu_env, Wi_env, W_env,
                cls_W, cls_b16)


def _tc_log_softmax2(l0_ref, l1_ref, o0_ref, o1_ref):
    a = l0_ref[...]
    b = l1_ref[...]
    m = jnp.maximum(a, b)
    lse = m + jnp.log(jnp.exp(a - m) + jnp.exp(b - m))
    o0_ref[...] = a - lse
    o1_ref[...] = b - lse


def kernel(users_id, items_id, envs_id, alpha, Wu_inv, Wi_inv, Wu_env, Wi_env,
           W_env, cls_W, cls_b):
    del alpha
    uid = users_id.astype(jnp.int32).reshape(NW * NCH, CH)
    iid = items_id.astype(jnp.int32).reshape(NW * NCH, CH)
    eid = envs_id.astype(jnp.int32).reshape(NW * NCH, CH)
    cls_b16 = jnp.zeros((L,), jnp.float32).at[:2].set(cls_b.astype(jnp.float32))

    inv_score, env_score, l0, l1 = _sc_scores(
        uid, iid, eid,
        Wu_inv.reshape(NBLK, 128), Wi_inv.reshape(NBLK, 128),
        Wu_env.reshape(NBLK, 128), Wi_env.reshape(NBLK, 128),
        W_env, cls_W.astype(jnp.float32), cls_b16)

    o0, o1 = pl.pallas_call(
        _tc_log_softmax2,
        out_shape=[jax.ShapeDtypeStruct((128, 128), jnp.float32)] * 2,
    )(l0.reshape(128, 128), l1.reshape(128, 128))

    env_outputs = jnp.stack([o0.reshape(-1), o1.reshape(-1)], axis=-1)
    return inv_score, env_score, env_outputs


# speedup vs baseline: 1.1358x; 1.1338x over previous
import functools

import jax
import jax.numpy as jnp
from jax import lax
from jax.experimental import pallas as pl
from jax.experimental.pallas import tpu as pltpu
from jax.experimental.pallas import tpu_sc as plsc

F = 32
L = 16
NC, NS = 2, 16
NW = NC * NS
B = 16384
ROWS_W = B // NW
CH = 64
NCH = ROWS_W // CH
RPB = 128 // F
NBLK = 1000000 // RPB


def _sc_scores(uid, iid, eid, Wu_inv, Wi_inv, Wu_env, Wi_env, W_env, cls_W,
               cls_b16):
    mesh = plsc.VectorSubcoreMesh(core_axis_name="c", subcore_axis_name="s")
    out_type = [jax.ShapeDtypeStruct((B,), jnp.float32) for _ in range(4)]
    scratch_types = [
        pltpu.VMEM((NCH, CH), jnp.int32),
        pltpu.VMEM((NCH, CH), jnp.int32),
        pltpu.VMEM((NCH, CH), jnp.int32),
        pltpu.VMEM((NCH, CH), jnp.int32),
        pltpu.VMEM((NCH, CH), jnp.int32),
        pltpu.VMEM((2, CH, 128), jnp.float32),
        pltpu.VMEM((2, CH, 128), jnp.float32),
        pltpu.VMEM((2, CH, 128), jnp.float32),
        pltpu.VMEM((2, CH, 128), jnp.float32),
        pltpu.VMEM((ROWS_W,), jnp.float32),
        pltpu.VMEM((ROWS_W,), jnp.float32),
        pltpu.VMEM((ROWS_W,), jnp.float32),
        pltpu.VMEM((ROWS_W,), jnp.float32),
        pltpu.VMEM((2, F), jnp.float32),
        pltpu.VMEM((2, F), jnp.float32),
        pltpu.VMEM((L,), jnp.float32),
        pltpu.SemaphoreType.DMA,
        pltpu.SemaphoreType.DMA,
    ]

    @functools.partial(pl.kernel, mesh=mesh, out_type=out_type,
                       scratch_types=scratch_types,
                       compiler_params=pltpu.CompilerParams(
                           needs_layout_passes=False,
                           use_tc_tiling_on_sc=True))
    def body(uid_h, iid_h, eid_h, wui_h, wii_h, wue_h, wie_h, wev_h, clsw_h,
             clsb_h, o_inv_h, o_env_h, o_l0_h, o_l1_h,
             ids_u, ids_i, ids_e, blk_u, blk_i,
             d_ui, d_ii, d_ue, d_ie,
             res_inv, res_env, res_l0, res_l1, clsw_v, wenv_v, clsb_v,
             sem0, sem1):
        wid = lax.axis_index("s") * NC + lax.axis_index("c")
        base_row = wid * NCH
        pltpu.sync_copy(uid_h.at[pl.ds(base_row, NCH)], ids_u)
        pltpu.sync_copy(iid_h.at[pl.ds(base_row, NCH)], ids_i)
        pltpu.sync_copy(eid_h.at[pl.ds(base_row, NCH)], ids_e)
        pltpu.sync_copy(clsw_h, clsw_v)
        pltpu.sync_copy(wev_h, wenv_v)
        pltpu.sync_copy(clsb_h, clsb_v)

        for j in range(NCH):
            def gen(kb, _, j=j):
                sl = pl.ds(kb * L, L)
                blk_u[j, sl] = ids_u[j, sl] >> 2
                blk_i[j, sl] = ids_i[j, sl] >> 2
                return 0
            lax.fori_loop(0, CH // L, gen, 0)

        sems = [sem0, sem1]
        copies = [None] * NCH

        def fire(j):
            s = j % 2
            copies[j] = [
                pltpu.async_copy(wui_h.at[blk_u.at[j]], d_ui.at[s], sems[s]),
                pltpu.async_copy(wii_h.at[blk_i.at[j]], d_ii.at[s], sems[s]),
                pltpu.async_copy(wue_h.at[blk_u.at[j]], d_ue.at[s], sems[s]),
                pltpu.async_copy(wie_h.at[blk_i.at[j]], d_ie.at[s], sems[s]),
            ]

        fire(0)
        fire(1)

        cw = [clsw_v[0, pl.ds(0, L)], clsw_v[0, pl.ds(L, L)],
              clsw_v[1, pl.ds(0, L)], clsw_v[1, pl.ds(L, L)]]
        we = [wenv_v[0, pl.ds(0, L)], wenv_v[0, pl.ds(L, L)],
              wenv_v[1, pl.ds(0, L)], wenv_v[1, pl.ds(L, L)]]
        w0 = [cw[f // L][f % L] for f in range(F)]
        w1 = [cw[2 + f // L][f % L] for f in range(F)]
        we0 = [we[f // L][f % L] for f in range(F)]
        we1 = [we[2 + f // L][f % L] for f in range(F)]
        vb = clsb_v[...]
        b0 = vb[0]
        b1 = vb[1]
        lanes = lax.iota(jnp.int32, L)
        zero = jnp.zeros((L,), jnp.float32)

        for j in range(NCH):
            for c in copies[j]:
                c.wait()
            s = j % 2
            vu = d_ui.at[s]
            vi = d_ii.at[s]
            vue = d_ue.at[s]
            vie = d_ie.at[s]

            def kblock(kb, _, j=j, s=s, vu=vu, vi=vi, vue=vue, vie=vie):
                eb = kb * L
                sl = pl.ds(eb, L)
                env_is0 = ids_e[j, sl] == 0
                ucol = (ids_u[j, sl] & 3) << 5
                icol = (ids_i[j, sl] & 3) << 5
                rows = eb + lanes
                acc_inv = zero
                acc_env = zero
                acc_l0 = zero
                acc_l1 = zero
                for f in range(F):
                    cu = ucol + f
                    ci = icol + f
                    gu = plsc.load_gather(vu, [rows, cu])
                    gi = plsc.load_gather(vi, [rows, ci])
                    p = gu * gi
                    acc_inv = acc_inv + p
                    acc_l0 = acc_l0 + p * w0[f]
                    acc_l1 = acc_l1 + p * w1[f]
                    eu = plsc.load_gather(vue, [rows, cu])
                    ei = plsc.load_gather(vie, [rows, ci])
                    ee = jnp.where(env_is0, we0[f], we1[f])
                    acc_env = acc_env + eu * ei * ee
                out = pl.ds(j * CH + eb, L)
                res_inv[out] = acc_inv
                res_env[out] = acc_inv + acc_env
                res_l0[out] = acc_l0 + b0
                res_l1[out] = acc_l1 + b1
                return 0

            lax.fori_loop(0, CH // L, kblock, 0)
            if j + 2 < NCH:
                fire(j + 2)

        base = wid * ROWS_W
        pltpu.sync_copy(res_inv, o_inv_h.at[pl.ds(base, ROWS_W)])
        pltpu.sync_copy(res_env, o_env_h.at[pl.ds(base, ROWS_W)])
        pltpu.sync_copy(res_l0, o_l0_h.at[pl.ds(base, ROWS_W)])
        pltpu.sync_copy(res_l1, o_l1_h.at[pl.ds(base, ROWS_W)])

    return body(uid, iid, eid, Wu_inv, Wi_inv, Wu_env, Wi_env, W_env,
                cls_W, cls_b16)


_KL = 2048
_NSTEP = -(-1000000 // _KL)


def _tc_repack(a_ref, b_ref, c_ref, d_ref, oa_ref, ob_ref, oc_ref, od_ref):
    for src, dst in ((a_ref, oa_ref), (b_ref, ob_ref), (c_ref, oc_ref),
                     (d_ref, od_ref)):
        x3 = src[...].T.reshape(_KL // RPB, RPB, F)
        for q in range(RPB):
            dst[:, q * F:(q + 1) * F] = x3[:, q, :]


def _repack_tables(wu_inv_t, wi_inv_t, wu_env_t, wi_env_t):
    in_spec = pl.BlockSpec((F, _KL), lambda g: (0, g))
    out_spec = pl.BlockSpec((_KL // RPB, 128), lambda g: (g, 0))
    return pl.pallas_call(
        _tc_repack,
        grid=(_NSTEP,),
        in_specs=[in_spec] * 4,
        out_specs=[out_spec] * 4,
        out_shape=[jax.ShapeDtypeStruct((NBLK, 128), jnp.float32)] * 4,
        compiler_params=pltpu.CompilerParams(
            dimension_semantics=("arbitrary",)),
    )(wu_inv_t, wi_inv_t, wu_env_t, wi_env_t)


def _tc_log_softmax2(l0_ref, l1_ref, o0_ref, o1_ref):
    a = l0_ref[...]
    b = l1_ref[...]
    m = jnp.maximum(a, b)
    lse = m + jnp.log(jnp.exp(a - m) + jnp.exp(b - m))
    o0_ref[...] = a - lse
    o1_ref[...] = b - lse


def kernel(users_id, items_id, envs_id, alpha, Wu_inv, Wi_inv, Wu_env, Wi_env,
           W_env, cls_W, cls_b):
    del alpha
    uid = users_id.astype(jnp.int32).reshape(NW * NCH, CH)
    iid = items_id.astype(jnp.int32).reshape(NW * NCH, CH)
    eid = envs_id.astype(jnp.int32).reshape(NW * NCH, CH)
    cls_b16 = jnp.zeros((L,), jnp.float32).at[:2].set(cls_b.astype(jnp.float32))

    rui, rii, rue, rie = _repack_tables(Wu_inv.T, Wi_inv.T, Wu_env.T,
                                        Wi_env.T)
    inv_score, env_score, l0, l1 = _sc_scores(
        uid, iid, eid, rui, rii, rue, rie,
        W_env, cls_W.astype(jnp.float32), cls_b16)

    o0, o1 = pl.pallas_call(
        _tc_log_softmax2,
        out_shape=[jax.ShapeDtypeStruct((128, 128), jnp.float32)] * 2,
    )(l0.reshape(128, 128), l1.reshape(128, 128))

    env_outputs = jnp.stack([o0.reshape(-1), o1.reshape(-1)], axis=-1)
    return inv_score, env_score, env_outputs


# speedup vs baseline: 1.5969x; 1.4060x over previous
import functools

import jax
import jax.numpy as jnp
from jax import lax
from jax.experimental import pallas as pl
from jax.experimental.pallas import tpu as pltpu
from jax.experimental.pallas import tpu_sc as plsc

F = 32
L = 16
NC, NS = 2, 16
NW = NC * NS
B = 16384
ROWS_W = B // NW
CH = 64
NCH = ROWS_W // CH
RPB = 128 // F
NBLK = 1000000 // RPB


def _sc_scores(uid, iid, eid, Wu_inv, Wi_inv, Wu_env, Wi_env, W_env, cls_W,
               cls_b16):
    mesh = plsc.VectorSubcoreMesh(core_axis_name="c", subcore_axis_name="s")
    out_type = [jax.ShapeDtypeStruct((B,), jnp.float32) for _ in range(4)]
    scratch_types = [
        pltpu.VMEM((NCH, CH), jnp.int32),
        pltpu.VMEM((NCH, CH), jnp.int32),
        pltpu.VMEM((NCH, CH), jnp.int32),
        pltpu.VMEM((NCH, CH), jnp.int32),
        pltpu.VMEM((NCH, CH), jnp.int32),
        pltpu.VMEM((2, CH, 128), jnp.float32),
        pltpu.VMEM((2, CH, 128), jnp.float32),
        pltpu.VMEM((2, CH, 128), jnp.float32),
        pltpu.VMEM((2, CH, 128), jnp.float32),
        pltpu.VMEM((ROWS_W,), jnp.float32),
        pltpu.VMEM((ROWS_W,), jnp.float32),
        pltpu.VMEM((ROWS_W,), jnp.float32),
        pltpu.VMEM((ROWS_W,), jnp.float32),
        pltpu.VMEM((2, F), jnp.float32),
        pltpu.VMEM((2, F), jnp.float32),
        pltpu.VMEM((L,), jnp.float32),
        pltpu.SemaphoreType.DMA,
        pltpu.SemaphoreType.DMA,
    ]

    @functools.partial(pl.kernel, mesh=mesh, out_type=out_type,
                       scratch_types=scratch_types,
                       compiler_params=pltpu.CompilerParams(
                           needs_layout_passes=False,
                           use_tc_tiling_on_sc=True))
    def body(uid_h, iid_h, eid_h, wui_h, wii_h, wue_h, wie_h, wev_h, clsw_h,
             clsb_h, o_inv_h, o_env_h, o_l0_h, o_l1_h,
             ids_u, ids_i, ids_e, blk_u, blk_i,
             d_ui, d_ii, d_ue, d_ie,
             res_inv, res_env, res_l0, res_l1, clsw_v, wenv_v, clsb_v,
             sem0, sem1):
        wid = lax.axis_index("s") * NC + lax.axis_index("c")
        base_row = wid * NCH
        pltpu.sync_copy(uid_h.at[pl.ds(base_row, NCH)], ids_u)
        pltpu.sync_copy(iid_h.at[pl.ds(base_row, NCH)], ids_i)
        pltpu.sync_copy(eid_h.at[pl.ds(base_row, NCH)], ids_e)
        pltpu.sync_copy(clsw_h, clsw_v)
        pltpu.sync_copy(wev_h, wenv_v)
        pltpu.sync_copy(clsb_h, clsb_v)

        for j in range(NCH):
            def gen(kb, _, j=j):
                sl = pl.ds(kb * L, L)
                uv = ids_u[j, sl]
                iv = ids_i[j, sl]
                blk_u[j, sl] = ((uv >> 11) << 9) + (uv & 511)
                blk_i[j, sl] = ((iv >> 11) << 9) + (iv & 511)
                return 0
            lax.fori_loop(0, CH // L, gen, 0)

        sems = [sem0, sem1]
        copies = [None] * NCH

        def fire(j):
            s = j % 2
            copies[j] = [
                pltpu.async_copy(wui_h.at[blk_u.at[j]], d_ui.at[s], sems[s]),
                pltpu.async_copy(wii_h.at[blk_i.at[j]], d_ii.at[s], sems[s]),
                pltpu.async_copy(wue_h.at[blk_u.at[j]], d_ue.at[s], sems[s]),
                pltpu.async_copy(wie_h.at[blk_i.at[j]], d_ie.at[s], sems[s]),
            ]

        fire(0)
        fire(1)

        cw = [clsw_v[0, pl.ds(0, L)], clsw_v[0, pl.ds(L, L)],
              clsw_v[1, pl.ds(0, L)], clsw_v[1, pl.ds(L, L)]]
        we = [wenv_v[0, pl.ds(0, L)], wenv_v[0, pl.ds(L, L)],
              wenv_v[1, pl.ds(0, L)], wenv_v[1, pl.ds(L, L)]]
        w0 = [cw[f // L][f % L] for f in range(F)]
        w1 = [cw[2 + f // L][f % L] for f in range(F)]
        we0 = [we[f // L][f % L] for f in range(F)]
        we1 = [we[2 + f // L][f % L] for f in range(F)]
        vb = clsb_v[...]
        b0 = vb[0]
        b1 = vb[1]
        lanes = lax.iota(jnp.int32, L)
        zero = jnp.zeros((L,), jnp.float32)

        for j in range(NCH):
            for c in copies[j]:
                c.wait()
            s = j % 2
            vu = d_ui.at[s]
            vi = d_ii.at[s]
            vue = d_ue.at[s]
            vie = d_ie.at[s]

            def kblock(kb, _, j=j, s=s, vu=vu, vi=vi, vue=vue, vie=vie):
                eb = kb * L
                sl = pl.ds(eb, L)
                env_is0 = ids_e[j, sl] == 0
                ucol = ((ids_u[j, sl] >> 9) & 3) << 5
                icol = ((ids_i[j, sl] >> 9) & 3) << 5
                rows = eb + lanes
                acc_inv = zero
                acc_env = zero
                acc_l0 = zero
                acc_l1 = zero
                for f in range(F):
                    cu = ucol + f
                    ci = icol + f
                    gu = plsc.load_gather(vu, [rows, cu])
                    gi = plsc.load_gather(vi, [rows, ci])
                    p = gu * gi
                    acc_inv = acc_inv + p
                    acc_l0 = acc_l0 + p * w0[f]
                    acc_l1 = acc_l1 + p * w1[f]
                    eu = plsc.load_gather(vue, [rows, cu])
                    ei = plsc.load_gather(vie, [rows, ci])
                    ee = jnp.where(env_is0, we0[f], we1[f])
                    acc_env = acc_env + eu * ei * ee
                out = pl.ds(j * CH + eb, L)
                res_inv[out] = acc_inv
                res_env[out] = acc_inv + acc_env
                res_l0[out] = acc_l0 + b0
                res_l1[out] = acc_l1 + b1
                return 0

            lax.fori_loop(0, CH // L, kblock, 0)
            if j + 2 < NCH:
                fire(j + 2)

        base = wid * ROWS_W
        pltpu.sync_copy(res_inv, o_inv_h.at[pl.ds(base, ROWS_W)])
        pltpu.sync_copy(res_env, o_env_h.at[pl.ds(base, ROWS_W)])
        pltpu.sync_copy(res_l0, o_l0_h.at[pl.ds(base, ROWS_W)])
        pltpu.sync_copy(res_l1, o_l1_h.at[pl.ds(base, ROWS_W)])

    return body(uid, iid, eid, Wu_inv, Wi_inv, Wu_env, Wi_env, W_env,
                cls_W, cls_b16)


_KL = 2048
_NSTEP = -(-1000000 // _KL)


def _tc_repack(a_ref, b_ref, c_ref, d_ref, oa_ref, ob_ref, oc_ref, od_ref):
    eye = jnp.eye(F, dtype=jnp.float32)
    rpb = _KL // RPB
    for src, dst in ((a_ref, oa_ref), (b_ref, ob_ref), (c_ref, oc_ref),
                     (d_ref, od_ref)):
        xt = lax.dot_general(src[...], eye, (((0,), (0,)), ((), ())),
                             preferred_element_type=jnp.float32)
        for q in range(RPB):
            dst[:, q * F:(q + 1) * F] = xt[q * rpb:(q + 1) * rpb, :]


def _repack_tables(wu_inv_t, wi_inv_t, wu_env_t, wi_env_t):
    in_spec = pl.BlockSpec((F, _KL), lambda g: (0, g))
    out_spec = pl.BlockSpec((_KL // RPB, 128), lambda g: (g, 0))
    return pl.pallas_call(
        _tc_repack,
        grid=(_NSTEP,),
        in_specs=[in_spec] * 4,
        out_specs=[out_spec] * 4,
        out_shape=[jax.ShapeDtypeStruct((_NSTEP * (_KL // RPB), 128),
                                        jnp.float32)] * 4,
        compiler_params=pltpu.CompilerParams(
            dimension_semantics=("arbitrary",)),
    )(wu_inv_t, wi_inv_t, wu_env_t, wi_env_t)


def _tc_log_softmax2(l0_ref, l1_ref, o0_ref, o1_ref):
    a = l0_ref[...]
    b = l1_ref[...]
    m = jnp.maximum(a, b)
    lse = m + jnp.log(jnp.exp(a - m) + jnp.exp(b - m))
    o0_ref[...] = a - lse
    o1_ref[...] = b - lse


def kernel(users_id, items_id, envs_id, alpha, Wu_inv, Wi_inv, Wu_env, Wi_env,
           W_env, cls_W, cls_b):
    del alpha
    uid = users_id.astype(jnp.int32).reshape(NW * NCH, CH)
    iid = items_id.astype(jnp.int32).reshape(NW * NCH, CH)
    eid = envs_id.astype(jnp.int32).reshape(NW * NCH, CH)
    cls_b16 = jnp.zeros((L,), jnp.float32).at[:2].set(cls_b.astype(jnp.float32))

    rui, rii, rue, rie = _repack_tables(Wu_inv.T, Wi_inv.T, Wu_env.T,
                                        Wi_env.T)
    inv_score, env_score, l0, l1 = _sc_scores(
        uid, iid, eid, rui, rii, rue, rie,
        W_env, cls_W.astype(jnp.float32), cls_b16)

    o0, o1 = pl.pallas_call(
        _tc_log_softmax2,
        out_shape=[jax.ShapeDtypeStruct((128, 128), jnp.float32)] * 2,
    )(l0.reshape(128, 128), l1.reshape(128, 128))

    env_outputs = jnp.stack([o0.reshape(-1), o1.reshape(-1)], axis=-1)
    return inv_score, env_score, env_outputs
